# baseline (speedup 1.0000x reference)
import functools

import jax
import jax.numpy as jnp
from jax import lax
from jax.experimental import pallas as pl
from jax.experimental.pallas import tpu as pltpu
from jax.experimental.pallas import tpu_sc as plsc

B = 16384
NF = 8
EMB = 32
H1 = 8

NC = 2
NS = 16
L = 16
NW = NC * NS
RPW = B // NW
OUT_W = 2 * NF
RPS = NF * 256 // NS
TBL = NF * 256 * 2

RW1, RB1, RW2, RB2, RLEN = 0, 256, 264, 280, 288
PB1 = EMB * H1 * L
PW2 = PB1 + H1 * L
PB2 = PW2 + H1 * 2 * L
PLEN = PB2 + 2 * L

_sc_mesh = plsc.VectorSubcoreMesh(core_axis_name="c", subcore_axis_name="s")


@functools.partial(
    pl.kernel,
    mesh=_sc_mesh,
    out_type=jax.ShapeDtypeStruct((B * OUT_W,), jnp.float32),
    scratch_types=[
        pltpu.VMEM((RPS * EMB,), jnp.float32),
        pltpu.VMEM((RLEN,), jnp.int32),
        pltpu.VMEM((PLEN,), jnp.float32),
        pltpu.VMEM((RPS * 2,), jnp.float32),
        pltpu.VMEM((TBL,), jnp.float32),
        pltpu.VMEM((NF, RPW), jnp.int32),
        pltpu.VMEM((RPW * OUT_W,), jnp.float32),
        pltpu.VMEM_SHARED((TBL,), jnp.float32),
        pltpu.SemaphoreType.DMA,
    ],
    compiler_params=pltpu.CompilerParams(needs_layout_passes=False),
)
def _sc_all(e1, e2, e3, e4, e5, e6, e7, e8,
            w11, w21, w31, w41, wp1, b11, b21, b31, b41, bp1,
            wa2, ba2, wp2, bp2,
            i1, i2, i3, i4, i5, i6, i7, i8, out_hbm,
            e_v, raw_v, p_v, tloc_v, t_v, idx_v, out_v, shared, sem):
    s = lax.axis_index("s")
    cax = lax.axis_index("c")
    wid = s * NC + cax
    base = wid * RPW

    idx_hbms = (i1, i2, i3, i4, i5, i6, i7, i8)
    idx_copies = [
        pltpu.async_copy(idx_hbms[n].at[pl.ds(base, RPW)], idx_v.at[n], sem)
        for n in range(NF)
    ]

    f = s // 2
    lr0 = (s % 2) * RPS
    embs = (e1, e2, e3, e4, e5, e6, e7, e8)
    w1s = (w11, w21, w31, w41, wp1, wp1, wp1, wp1)
    b1s = (b11, b21, b31, b41, bp1, bp1, bp1, bp1)
    w2s = (wa2, wa2, wa2, wa2, wp2, wp2, wp2, wp2)
    b2s = (ba2, ba2, ba2, ba2, bp2, bp2, bp2, bp2)
    for ff in range(NF):
        @pl.when(f == ff)
        def _(ff=ff):
            pltpu.sync_copy(embs[ff].at[pl.ds(lr0 * EMB, RPS * EMB)], e_v)
            pltpu.sync_copy(w1s[ff], raw_v.at[pl.ds(RW1, 256)])
            pltpu.sync_copy(b1s[ff], raw_v.at[pl.ds(RB1, 8)])
            pltpu.sync_copy(w2s[ff], raw_v.at[pl.ds(RW2, 16)])
            pltpu.sync_copy(b2s[ff], raw_v.at[pl.ds(RB2, 2)])

    def splat_range(m0, cnt, dst):
        def sbody(m, _):
            w = plsc.load_gather(raw_v, [jnp.full((L,), 0, jnp.int32) + m0 + m])
            plsc.store_scatter(p_v, [lax.iota(jnp.int32, L) + dst + m * L],
                               plsc.bitcast(w, jnp.float32))
            return 0
        lax.fori_loop(0, cnt, sbody, 0)

    splat_range(RW1, 256, 0)
    splat_range(RB1, 8, PB1)
    splat_range(RW2, 16, PW2)
    splat_range(RB2, 2, PB2)

    lane = lax.iota(jnp.int32, L)
    lane2 = lane * 2
    lane_e = lane * EMB

    def tbody(c, _):
        ebase = lane_e + c * (L * EMB)
        cols = [plsc.load_gather(e_v, [ebase + k]) for k in range(EMB)]
        t0 = p_v[pl.ds(PB2, L)]
        t1 = p_v[pl.ds(PB2 + L, L)]
        for j in range(H1):
            acc = p_v[pl.ds(PB1 + j * L, L)]
            for k in range(EMB):
                acc = acc + cols[k] * p_v[pl.ds((k * H1 + j) * L, L)]
            h = jnp.maximum(acc, 0.0)
            t0 = t0 + h * p_v[pl.ds(PW2 + (j * 2) * L, L)]
            t1 = t1 + h * p_v[pl.ds(PW2 + (j * 2 + 1) * L, L)]
        pos = c * (2 * L) + lane2
        plsc.store_scatter(tloc_v, [pos], jnp.maximum(t0, 0.0))
        plsc.store_scatter(tloc_v, [pos + 1], jnp.maximum(t1, 0.0))
        return 0

    lax.fori_loop(0, RPS // L, tbody, 0)

    pltpu.sync_copy(tloc_v, shared.at[pl.ds(s * RPS * 2, RPS * 2)])
    plsc.subcore_barrier()
    pltpu.sync_copy(shared, t_v)

    for c in idx_copies:
        c.wait()

    nchunks = RPW // L
    for n in range(NF):
        pos_n = lane * OUT_W + 2 * n

        def gbody(c, _, n=n, pos_n=pos_n):
            g2 = (idx_v[n, pl.ds(c * L, L)] + n * 256) * 2
            v0 = plsc.load_gather(t_v, [g2])
            v1 = plsc.load_gather(t_v, [g2 + 1])
            pos = pos_n + c * (L * OUT_W)
            plsc.store_scatter(out_v, [pos], v0)
            plsc.store_scatter(out_v, [pos + 1], v1)
            return 0

        lax.fori_loop(0, nchunks, gbody, 0, unroll=4)

    pltpu.sync_copy(out_v, out_hbm.at[pl.ds(base * OUT_W, RPW * OUT_W)])


def kernel(idx_a1, idx_a2, idx_a3, idx_a4, idx_p1, idx_p2, idx_p3, idx_p4,
           emb_a1, emb_a2, emb_a3, emb_a4, emb_p1, emb_p2, emb_p3, emb_p4,
           Wa1_1, ba1_1, Wa2_1, ba2_1, Wa3_1, ba3_1, Wa4_1, ba4_1,
           Wp1_1, bp1_1, Wa1_2, ba1_2, Wp1_2, bp1_2):
    bits = lambda a: a.reshape(-1).view(jnp.int32)
    out_flat = _sc_all(
        emb_a1.reshape(-1), emb_a2.reshape(-1), emb_a3.reshape(-1),
        emb_a4.reshape(-1), emb_p1.reshape(-1), emb_p2.reshape(-1),
        emb_p3.reshape(-1), emb_p4.reshape(-1),
        bits(Wa1_1), bits(Wa2_1), bits(Wa3_1), bits(Wa4_1), bits(Wp1_1),
        bits(ba1_1), bits(ba2_1), bits(ba3_1), bits(ba4_1), bits(bp1_1),
        bits(Wa1_2), bits(ba1_2), bits(Wp1_2), bits(bp1_2),
        idx_a1, idx_a2, idx_a3, idx_a4, idx_p1, idx_p2, idx_p3, idx_p4)
    return out_flat.reshape(B, OUT_W)


# speedup vs baseline: 1.2682x; 1.2682x over previous
import functools

import jax
import jax.numpy as jnp
from jax import lax
from jax.experimental import pallas as pl
from jax.experimental.pallas import tpu as pltpu
from jax.experimental.pallas import tpu_sc as plsc

B = 16384
NF = 8
EMB = 32
H1 = 8

NC = 2
NS = 16
L = 16
NW = NC * NS
RPW = B // NW
OUT_W = 2 * NF
RPS = NF * 256 // NS
TBL = NF * 256 * 2

PB1 = EMB * H1 * L
PW2 = PB1 + H1 * L
PB2 = PW2 + H1 * 2 * L
PLEN = PB2 + 2 * L

_sc_mesh = plsc.VectorSubcoreMesh(core_axis_name="c", subcore_axis_name="s")


@functools.partial(
    pl.kernel,
    mesh=_sc_mesh,
    out_type=jax.ShapeDtypeStruct((B * OUT_W,), jnp.float32),
    scratch_types=[
        pltpu.VMEM((RPS * EMB,), jnp.float32),
        pltpu.VMEM((PLEN,), jnp.float32),
        pltpu.VMEM((RPS * 2,), jnp.float32),
        pltpu.VMEM((TBL,), jnp.float32),
        pltpu.VMEM((NF, RPW), jnp.int32),
        pltpu.VMEM((RPW * OUT_W,), jnp.float32),
        pltpu.VMEM_SHARED((TBL,), jnp.float32),
        pltpu.SemaphoreType.DMA,
    ],
    compiler_params=pltpu.CompilerParams(needs_layout_passes=False),
)
def _sc_all(emb_hbm, p_hbm, i1, i2, i3, i4, i5, i6, i7, i8, out_hbm,
            e_v, p_v, tloc_v, t_v, idx_v, out_v, shared, sem):
    s = lax.axis_index("s")
    cax = lax.axis_index("c")
    wid = s * NC + cax
    base = wid * RPW

    idx_hbms = (i1, i2, i3, i4, i5, i6, i7, i8)
    idx_copies = [
        pltpu.async_copy(idx_hbms[n].at[pl.ds(base, RPW)], idx_v.at[n], sem)
        for n in range(NF)
    ]

    f = s // 2
    lr0 = (s % 2) * RPS
    pltpu.sync_copy(emb_hbm.at[f, pl.ds(lr0 * EMB, RPS * EMB)], e_v)
    pltpu.sync_copy(p_hbm.at[f], p_v)

    lane = lax.iota(jnp.int32, L)
    lane2 = lane * 2
    lane_e = lane * EMB

    def tbody(c, _):
        ebase = lane_e + c * (L * EMB)
        cols = [plsc.load_gather(e_v, [ebase + k]) for k in range(EMB)]
        t0 = p_v[pl.ds(PB2, L)]
        t1 = p_v[pl.ds(PB2 + L, L)]
        for j in range(H1):
            acc = p_v[pl.ds(PB1 + j * L, L)]
            for k in range(EMB):
                acc = acc + cols[k] * p_v[pl.ds((k * H1 + j) * L, L)]
            h = jnp.maximum(acc, 0.0)
            t0 = t0 + h * p_v[pl.ds(PW2 + (j * 2) * L, L)]
            t1 = t1 + h * p_v[pl.ds(PW2 + (j * 2 + 1) * L, L)]
        pos = c * (2 * L) + lane2
        plsc.store_scatter(tloc_v, [pos], jnp.maximum(t0, 0.0))
        plsc.store_scatter(tloc_v, [pos + 1], jnp.maximum(t1, 0.0))
        return 0

    lax.fori_loop(0, RPS // L, tbody, 0)

    pltpu.sync_copy(tloc_v, shared.at[pl.ds(s * RPS * 2, RPS * 2)])
    plsc.subcore_barrier()
    pltpu.sync_copy(shared, t_v)

    for c in idx_copies:
        c.wait()

    nchunks = RPW // L
    for n in range(NF):
        pos_n = lane * OUT_W + 2 * n

        def gbody(c, _, n=n, pos_n=pos_n):
            g2 = (idx_v[n, pl.ds(c * L, L)] + n * 256) * 2
            v0 = plsc.load_gather(t_v, [g2])
            v1 = plsc.load_gather(t_v, [g2 + 1])
            pos = pos_n + c * (L * OUT_W)
            plsc.store_scatter(out_v, [pos], v0)
            plsc.store_scatter(out_v, [pos + 1], v1)
            return 0

        lax.fori_loop(0, nchunks, gbody, 0, unroll=4)

    pltpu.sync_copy(out_v, out_hbm.at[pl.ds(base * OUT_W, RPW * OUT_W)])


def kernel(idx_a1, idx_a2, idx_a3, idx_a4, idx_p1, idx_p2, idx_p3, idx_p4,
           emb_a1, emb_a2, emb_a3, emb_a4, emb_p1, emb_p2, emb_p3, emb_p4,
           Wa1_1, ba1_1, Wa2_1, ba2_1, Wa3_1, ba3_1, Wa4_1, ba4_1,
           Wp1_1, bp1_1, Wa1_2, ba1_2, Wp1_2, bp1_2):
    emb = jnp.stack([emb_a1, emb_a2, emb_a3, emb_a4,
                     emb_p1, emb_p2, emb_p3, emb_p4])
    emb_flat = emb.reshape(NF, 256 * EMB)

    w1 = jnp.stack([Wa1_1, Wa2_1, Wa3_1, Wa4_1,
                    Wp1_1, Wp1_1, Wp1_1, Wp1_1])
    b1 = jnp.stack([ba1_1, ba2_1, ba3_1, ba4_1,
                    bp1_1, bp1_1, bp1_1, bp1_1])
    w2 = jnp.stack([Wa1_2, Wa1_2, Wa1_2, Wa1_2,
                    Wp1_2, Wp1_2, Wp1_2, Wp1_2])
    b2 = jnp.stack([ba1_2, ba1_2, ba1_2, ba1_2,
                    bp1_2, bp1_2, bp1_2, bp1_2])
    p = jnp.concatenate([w1.reshape(NF, -1), b1,
                         w2.reshape(NF, -1), b2], axis=1)
    p_splat = jnp.repeat(p, L, axis=1)

    out_flat = _sc_all(emb_flat, p_splat,
                       idx_a1, idx_a2, idx_a3, idx_a4,
                       idx_p1, idx_p2, idx_p3, idx_p4)
    return out_flat.reshape(B, OUT_W)
